# baseline (speedup 1.0000x reference)
import functools

import jax
import jax.numpy as jnp
from jax import lax
from jax.experimental import pallas as pl
from jax.experimental.pallas import tpu as pltpu
from jax.experimental.pallas import tpu_sc as plsc

_NUM_CLASSES = 1000000
_FEAT_DIM = 64
_BATCH = 16384
_LAMBDA_C = 1.0

_NC = 2
_NS = 16
_NW = _NC * _NS
_ROWS_W = _BATCH // _NW
_CHUNK = 128
_NCHUNK = _ROWS_W // _CHUNK
_LANES = 16
_GROUPS = _FEAT_DIM // _LANES


def _cl_body(feat_hbm, lab_hbm, cent_hbm, out_hbm,
             idx_v, feat_v, rows_v, acc_v, sem):
    wid = lax.axis_index("s") * _NC + lax.axis_index("c")
    base = wid * _ROWS_W

    pltpu.sync_copy(lab_hbm.at[pl.ds(wid * _NCHUNK, _NCHUNK), :], idx_v)

    copies = []
    for c in range(_NCHUNK):
        copies.append(
            pltpu.async_copy(
                cent_hbm.at[idx_v.at[c]],
                rows_v.at[pl.ds(c * _CHUNK, _CHUNK), :],
                sem,
            )
        )
    pltpu.sync_copy(feat_hbm.at[pl.ds(base, _ROWS_W), :], feat_v)
    for cp in copies:
        cp.wait()

    def row_step(i, acc):
        for j in range(_GROUPS):
            f = feat_v[i, pl.ds(j * _LANES, _LANES)]
            ce = rows_v[i, pl.ds(j * _LANES, _LANES)]
            d = f - ce
            acc = acc + d * d
        return acc

    acc = lax.fori_loop(0, _ROWS_W, row_step,
                        jnp.zeros((_LANES,), jnp.float32))
    acc_v[...] = acc * (_LAMBDA_C / float(_BATCH * _FEAT_DIM))
    pltpu.sync_copy(acc_v, out_hbm.at[wid])


@jax.jit
def kernel(features, labels, centers):
    lab2d = labels.reshape(_NW * _NCHUNK, _CHUNK).astype(jnp.int32)
    mesh = plsc.VectorSubcoreMesh(core_axis_name="c", subcore_axis_name="s")
    partials = pl.kernel(
        _cl_body,
        mesh=mesh,
        compiler_params=pltpu.CompilerParams(use_tc_tiling_on_sc=False),
        out_type=jax.ShapeDtypeStruct((_NW, _LANES), jnp.float32),
        scratch_types=[
            pltpu.VMEM((_NCHUNK, _CHUNK), jnp.int32),
            pltpu.VMEM((_ROWS_W, _FEAT_DIM), jnp.float32),
            pltpu.VMEM((_ROWS_W, _FEAT_DIM), jnp.float32),
            pltpu.VMEM((_LANES,), jnp.float32),
            pltpu.SemaphoreType.DMA,
        ],
    )(features, lab2d, centers)
    return jnp.sum(partials)


# speedup vs baseline: 1.6844x; 1.6844x over previous
import jax
import jax.numpy as jnp
from jax import lax
from jax.experimental import pallas as pl
from jax.experimental.pallas import tpu as pltpu
from jax.experimental.pallas import tpu_sc as plsc

_NUM_CLASSES = 1000000
_FEAT_DIM = 64
_BATCH = 16384
_LAMBDA_C = 1.0

_NC = 2
_NS = 16
_NW = _NC * _NS
_ROWS_W = _BATCH // _NW
_CHUNK = 128
_NCHUNK = _ROWS_W // _CHUNK
_LANES = 16
_GROUPS = _FEAT_DIM // _LANES


def _cl_body(feat_hbm, lab_hbm, cent_hbm, out_hbm,
             lab_v, feat_a, feat_b, rows_v, acc_v,
             semf, sem0, sem1, sem2, sem3):
    wid = lax.axis_index("s") * _NC + lax.axis_index("c")
    base = wid * _ROWS_W
    row_sems = [sem0, sem1, sem2, sem3]
    fbufs = [feat_a, feat_b]

    pltpu.sync_copy(lab_hbm.at[pl.ds(base, _ROWS_W)], lab_v)

    fcps = [pltpu.async_copy(feat_hbm.at[pl.ds(base, _CHUNK), :],
                             feat_a, semf)]

    for c in range(_NCHUNK):
        def issue(g, carry, c=c):
            vec = lab_v[pl.ds(c * _CHUNK + g * _LANES, _LANES)]
            for l in range(_LANES):
                r = vec[l]
                pltpu.async_copy(
                    cent_hbm.at[pl.ds(r, 1), :],
                    rows_v.at[pl.ds(c * _CHUNK + g * _LANES + l, 1), :],
                    row_sems[c])
            return carry
        lax.fori_loop(0, _CHUNK // _LANES, issue, 0)

    acc = jnp.zeros((_LANES,), jnp.float32)
    for c in range(_NCHUNK):
        if c + 1 < _NCHUNK:
            fcps.append(
                pltpu.async_copy(
                    feat_hbm.at[pl.ds(base + (c + 1) * _CHUNK, _CHUNK), :],
                    fbufs[(c + 1) % 2], semf))
        fcps[c].wait()
        pltpu.make_async_copy(cent_hbm.at[pl.ds(0, _CHUNK), :],
                              rows_v.at[pl.ds(c * _CHUNK, _CHUNK), :],
                              row_sems[c]).wait()

        fbuf = fbufs[c % 2]

        def row_step(i, acc, c=c, fbuf=fbuf):
            for j in range(_GROUPS):
                f = fbuf[i, pl.ds(j * _LANES, _LANES)]
                ce = rows_v[c * _CHUNK + i, pl.ds(j * _LANES, _LANES)]
                d = f - ce
                acc = acc + d * d
            return acc

        acc = lax.fori_loop(0, _CHUNK, row_step, acc)

    acc_v[...] = acc * (_LAMBDA_C / float(_BATCH * _FEAT_DIM))
    pltpu.sync_copy(acc_v, out_hbm.at[wid])


@jax.jit
def kernel(features, labels, centers):
    mesh = plsc.VectorSubcoreMesh(core_axis_name="c", subcore_axis_name="s")
    partials = pl.kernel(
        _cl_body,
        mesh=mesh,
        out_type=jax.ShapeDtypeStruct((_NW, _LANES), jnp.float32),
        scratch_types=[
            pltpu.VMEM((_ROWS_W,), jnp.int32),
            pltpu.VMEM((_CHUNK, _FEAT_DIM), jnp.float32),
            pltpu.VMEM((_CHUNK, _FEAT_DIM), jnp.float32),
            pltpu.VMEM((_ROWS_W, _FEAT_DIM), jnp.float32),
            pltpu.VMEM((_LANES,), jnp.float32),
            pltpu.SemaphoreType.DMA,
            pltpu.SemaphoreType.DMA,
            pltpu.SemaphoreType.DMA,
            pltpu.SemaphoreType.DMA,
            pltpu.SemaphoreType.DMA,
        ],
    )(features, labels.astype(jnp.int32), centers)
    return jnp.sum(partials)
